# Initial kernel scaffold; baseline (speedup 1.0000x reference)
import jax
import jax.numpy as jnp
from jax.experimental import pallas as pl


def kernel(x, edge_index, edge_attr, W1_rel, b1_rel, W1_root, W2_rel, b2_rel, W2_root):
    raise NotImplementedError("write your pallas kernel here")


# speedup vs baseline: 8.3749x; 8.3749x over previous
import functools

import jax
import jax.numpy as jnp
from jax import lax
from jax.experimental import pallas as pl
from jax.experimental.pallas import tpu as pltpu
from jax.experimental.pallas import tpu_sc as plsc

N = 10000
E = 320000
D = 128
H = 512
C = 2

NC = 2
NS = 16
NW = NC * NS
K = 128
CHUNKS = -(-E // (NW * K))
EP = NW * K * CHUNKS
NP = 10240
ROWS_PER_SUBCORE = NP // NS
BN = 1024
GRID = NP // BN


def _seg1_body(x_hbm, src_hbm, dst_hbm, ew_hbm, zrows_hbm, out_hbm,
               src_v, dst_v, ew_v, rows_v, acc_sh, gsem):
  cid = lax.axis_index("c")
  sid = lax.axis_index("s")

  rsl = pl.ds(sid * ROWS_PER_SUBCORE, ROWS_PER_SUBCORE)
  pltpu.sync_copy(zrows_hbm, acc_sh.at[rsl])
  plsc.subcore_barrier()

  def chunk_body(c, carry):
    pltpu.sync_copy(src_hbm.at[cid, sid, c], src_v)
    pltpu.sync_copy(dst_hbm.at[cid, sid, c], dst_v)
    pltpu.sync_copy(ew_hbm.at[cid, sid, c], ew_v)
    pltpu.async_copy(x_hbm.at[src_v], rows_v, gsem).wait()

    def group_body(g, carry2):
      ewg = ew_v[pl.ds(g * 16, 16)]
      for j in range(16):
        e = g * 16 + j
        w = jnp.full((16,), ewg[j], dtype=jnp.float32)
        for jj in range(D // 16):
          sl = pl.ds(jj * 16, 16)
          rows_v[e, sl] = rows_v[e, sl] * w
      return carry2

    lax.fori_loop(0, K // 16, group_body, 0)

    pltpu.sync_copy(rows_v, acc_sh.at[dst_v], add=True)
    return carry

  lax.fori_loop(0, CHUNKS, chunk_body, 0)
  plsc.subcore_barrier()

  pltpu.sync_copy(acc_sh.at[rsl], out_hbm.at[cid, rsl])


def _make_seg1():
  mesh = plsc.VectorSubcoreMesh(core_axis_name="c", subcore_axis_name="s",
                                num_cores=NC, num_subcores=NS)
  return pl.kernel(
      _seg1_body,
      out_type=jax.ShapeDtypeStruct((NC, NP, D), jnp.float32),
      mesh=mesh,
      scratch_types=[
          pltpu.VMEM((K,), jnp.int32),
          pltpu.VMEM((K,), jnp.int32),
          pltpu.VMEM((K,), jnp.float32),
          pltpu.VMEM((K, D), jnp.float32),
          pltpu.VMEM_SHARED((NP, D), jnp.float32),
          pltpu.SemaphoreType.DMA,
      ],
  )


def _seg2_body(pqt_hbm, src_hbm, dst_hbm, ew_hbm, z_hbm, out_hbm,
               p0_v, p1_v, a0_v, a1_v, src_v, dst_v, ew_v):
  cid = lax.axis_index("c")
  sid = lax.axis_index("s")
  pltpu.sync_copy(pqt_hbm.at[0], p0_v)
  pltpu.sync_copy(pqt_hbm.at[1], p1_v)
  pltpu.sync_copy(z_hbm, a0_v)
  pltpu.sync_copy(z_hbm, a1_v)

  def chunk_body(c, carry):
    pltpu.sync_copy(src_hbm.at[cid, sid, c], src_v)
    pltpu.sync_copy(dst_hbm.at[cid, sid, c], dst_v)
    pltpu.sync_copy(ew_hbm.at[cid, sid, c], ew_v)

    def group_body(g, carry2):
      sl = pl.ds(g * 16, 16)
      sidx = src_v[sl]
      didx = dst_v[sl]
      eww = ew_v[sl]
      for pv, av in ((p0_v, a0_v), (p1_v, a1_v)):
        v = plsc.load_gather(pv, [sidx])
        plsc.addupdate_scatter(av, [didx], v * eww)
      return carry2

    lax.fori_loop(0, K // 16, group_body, 0)
    return carry

  lax.fori_loop(0, CHUNKS, chunk_body, 0)
  pltpu.sync_copy(a0_v, out_hbm.at[cid, sid, 0])
  pltpu.sync_copy(a1_v, out_hbm.at[cid, sid, 1])


def _make_seg2():
  mesh = plsc.VectorSubcoreMesh(core_axis_name="c", subcore_axis_name="s",
                                num_cores=NC, num_subcores=NS)
  return pl.kernel(
      _seg2_body,
      out_type=jax.ShapeDtypeStruct((NC, NS, C, NP), jnp.float32),
      mesh=mesh,
      scratch_types=[
          pltpu.VMEM((NP,), jnp.float32),
          pltpu.VMEM((NP,), jnp.float32),
          pltpu.VMEM((NP,), jnp.float32),
          pltpu.VMEM((NP,), jnp.float32),
          pltpu.VMEM((K,), jnp.int32),
          pltpu.VMEM((K,), jnp.int32),
          pltpu.VMEM((K,), jnp.float32),
      ],
      compiler_params=pltpu.CompilerParams(needs_layout_passes=False),
  )


def _dense_body(x_ref, agg_ref, w1rel_ref, w1root_ref, b1_ref, w2_ref,
                pqt_ref):
  agg = agg_ref[0] + agg_ref[1]
  s = jnp.dot(agg, w1rel_ref[...], preferred_element_type=jnp.float32)
  s += jnp.dot(x_ref[...], w1root_ref[...], preferred_element_type=jnp.float32)
  s += b1_ref[...]
  s = jnp.maximum(s, 0.0)
  pqt_ref[...] = lax.dot_general(
      w2_ref[...], s, (((0,), (1,)), ((), ())),
      preferred_element_type=jnp.float32)


def _final_body(agg2_ref, pqt_ref, b2_ref, out_ref):
  asum = agg2_ref[0, 0]
  for c in range(NC):
    for s2 in range(NS):
      if c or s2:
        asum += agg2_ref[c, s2]
  t = asum + pqt_ref[4:6, :] + b2_ref[...]
  m = jnp.max(t, axis=0, keepdims=True)
  lse = m + jnp.log(jnp.sum(jnp.exp(t - m), axis=0, keepdims=True))
  out_ref[...] = t - lse


def kernel(x, edge_index, edge_attr, W1_rel, b1_rel, W1_root, W2_rel, b2_rel,
           W2_root):
  src = edge_index[0]
  dst = edge_index[1]
  pad = EP - E
  src_r = jnp.pad(src, (0, pad)).reshape(NC, NS, CHUNKS, K)
  dst_r = jnp.pad(dst, (0, pad)).reshape(NC, NS, CHUNKS, K)
  ew_r = jnp.pad(edge_attr, (0, pad)).reshape(NC, NS, CHUNKS, K)
  zrows = jnp.zeros((ROWS_PER_SUBCORE, D), dtype=jnp.float32)
  zcol = jnp.zeros((NP,), dtype=jnp.float32)

  agg1 = _make_seg1()(x, src_r, dst_r, ew_r, zrows)

  w2pack = jnp.zeros((H, 8), dtype=jnp.float32)
  w2pack = w2pack.at[:, :C].set(W2_rel.T)
  w2pack = w2pack.at[:, 4:4 + C].set(W2_root.T)

  pqt = pl.pallas_call(
      _dense_body,
      grid=(GRID,),
      in_specs=[
          pl.BlockSpec((BN, D), lambda i: (i, 0)),
          pl.BlockSpec((NC, BN, D), lambda i: (0, i, 0)),
          pl.BlockSpec((D, H), lambda i: (0, 0)),
          pl.BlockSpec((D, H), lambda i: (0, 0)),
          pl.BlockSpec((1, H), lambda i: (0, 0)),
          pl.BlockSpec((H, 8), lambda i: (0, 0)),
      ],
      out_specs=pl.BlockSpec((8, BN), lambda i: (0, i)),
      out_shape=jax.ShapeDtypeStruct((8, NP), jnp.float32),
  )(x, agg1, W1_rel.T, W1_root.T, b1_rel.reshape(1, H), w2pack)

  agg2 = _make_seg2()(pqt, src_r, dst_r, ew_r, zcol)

  b2col = b2_rel.reshape(C, 1)
  outT = pl.pallas_call(
      _final_body,
      grid=(GRID,),
      in_specs=[
          pl.BlockSpec((NC, NS, C, BN), lambda i: (0, 0, 0, i)),
          pl.BlockSpec((8, BN), lambda i: (0, i)),
          pl.BlockSpec((C, 1), lambda i: (0, 0)),
      ],
      out_specs=pl.BlockSpec((C, BN), lambda i: (0, i)),
      out_shape=jax.ShapeDtypeStruct((C, N), jnp.float32),
  )(agg2, pqt, b2col)
  return outT.T


# speedup vs baseline: 9.8597x; 1.1773x over previous
import functools

import jax
import jax.numpy as jnp
from jax import lax
from jax.experimental import pallas as pl
from jax.experimental.pallas import tpu as pltpu
from jax.experimental.pallas import tpu_sc as plsc

N = 10000
E = 320000
D = 128
H = 512
C = 2

NC = 2
NS = 16
NW = NC * NS
K = 128
CHUNKS = 80
EP = NW * K * CHUNKS
NP = 10240
ROWS_PER_SUBCORE = NP // NS
BN = 1024
GRID = NP // BN


def _seg1_body(x_hbm, src_hbm, dst_hbm, ew_hbm, zrows_hbm, out_hbm,
               src_v, dst_v, ew_v, rows_v, acc_sh, gsem, isem):
  cid = lax.axis_index("c")
  sid = lax.axis_index("s")

  rsl = pl.ds(sid * ROWS_PER_SUBCORE, ROWS_PER_SUBCORE)
  pltpu.sync_copy(zrows_hbm, acc_sh.at[rsl])
  plsc.subcore_barrier()

  def load_idx(c, b):
    d1 = pltpu.async_copy(src_hbm.at[cid, sid, c], src_v.at[b], isem)
    d2 = pltpu.async_copy(dst_hbm.at[cid, sid, c], dst_v.at[b], isem)
    d3 = pltpu.async_copy(ew_hbm.at[cid, sid, c], ew_v.at[b], isem)
    d1.wait(); d2.wait(); d3.wait()

  def start_gather(b):
    return pltpu.async_copy(x_hbm.at[src_v.at[b]], rows_v.at[b], gsem)

  def process(c, b):
    def group_body(g, carry2):
      ewg = ew_v[b, pl.ds(g * 16, 16)]
      for j in range(16):
        e = g * 16 + j
        w = jnp.full((16,), ewg[j], dtype=jnp.float32)
        for jj in range(D // 16):
          sl = pl.ds(jj * 16, 16)
          rows_v[b, e, sl] = rows_v[b, e, sl] * w
      return carry2

    lax.fori_loop(0, K // 16, group_body, 0)
    pltpu.sync_copy(rows_v.at[b], acc_sh.at[dst_v.at[b]], add=True)

  load_idx(0, 0)
  start_gather(0)
  niter = CHUNKS // 2

  def pair_body(i, carry):
    c0 = i * 2
    load_idx(c0 + 1, 1)
    g1 = start_gather(1)
    pltpu.make_async_copy(x_hbm.at[src_v.at[0]], rows_v.at[0], gsem).wait()
    process(c0, 0)

    @pl.when(i + 1 < niter)
    def _prefetch():
      load_idx(c0 + 2, 0)
      start_gather(0)

    pltpu.make_async_copy(x_hbm.at[src_v.at[1]], rows_v.at[1], gsem).wait()
    process(c0 + 1, 1)
    return carry

  lax.fori_loop(0, niter, pair_body, 0)
  plsc.subcore_barrier()

  pltpu.sync_copy(acc_sh.at[rsl], out_hbm.at[cid, rsl])


def _make_seg1():
  mesh = plsc.VectorSubcoreMesh(core_axis_name="c", subcore_axis_name="s",
                                num_cores=NC, num_subcores=NS)
  return pl.kernel(
      _seg1_body,
      out_type=jax.ShapeDtypeStruct((NC, NP, D), jnp.float32),
      mesh=mesh,
      scratch_types=[
          pltpu.VMEM((2, K), jnp.int32),
          pltpu.VMEM((2, K), jnp.int32),
          pltpu.VMEM((2, K), jnp.float32),
          pltpu.VMEM((2, K, D), jnp.float32),
          pltpu.VMEM_SHARED((NP, D), jnp.float32),
          pltpu.SemaphoreType.DMA,
          pltpu.SemaphoreType.DMA,
      ],
  )


def _seg2_body(pqt_hbm, src_hbm, dst_hbm, ew_hbm, z_hbm, out_hbm,
               p0_v, p1_v, a0_v, a1_v, src_v, dst_v, ew_v, esem):
  cid = lax.axis_index("c")
  sid = lax.axis_index("s")
  d1 = pltpu.async_copy(src_hbm.at[cid, sid], src_v, esem)
  d2 = pltpu.async_copy(dst_hbm.at[cid, sid], dst_v, esem)
  d3 = pltpu.async_copy(ew_hbm.at[cid, sid], ew_v, esem)
  pltpu.sync_copy(pqt_hbm.at[0], p0_v)
  pltpu.sync_copy(pqt_hbm.at[1], p1_v)
  pltpu.sync_copy(z_hbm, a0_v)
  pltpu.sync_copy(z_hbm, a1_v)
  d1.wait(); d2.wait(); d3.wait()

  def chunk_body(c, carry):
    def group_body(g, carry2):
      sl = pl.ds(g * 16, 16)
      sidx = src_v[c, sl]
      didx = dst_v[c, sl]
      eww = ew_v[c, sl]
      for pv, av in ((p0_v, a0_v), (p1_v, a1_v)):
        v = plsc.load_gather(pv, [sidx])
        plsc.addupdate_scatter(av, [didx], v * eww)
      return carry2

    lax.fori_loop(0, K // 16, group_body, 0)
    return carry

  lax.fori_loop(0, CHUNKS, chunk_body, 0)
  pltpu.sync_copy(a0_v, out_hbm.at[cid, sid, 0])
  pltpu.sync_copy(a1_v, out_hbm.at[cid, sid, 1])


def _make_seg2():
  mesh = plsc.VectorSubcoreMesh(core_axis_name="c", subcore_axis_name="s",
                                num_cores=NC, num_subcores=NS)
  return pl.kernel(
      _seg2_body,
      out_type=jax.ShapeDtypeStruct((NC, NS, C, NP), jnp.float32),
      mesh=mesh,
      scratch_types=[
          pltpu.VMEM((NP,), jnp.float32),
          pltpu.VMEM((NP,), jnp.float32),
          pltpu.VMEM((NP,), jnp.float32),
          pltpu.VMEM((NP,), jnp.float32),
          pltpu.VMEM((CHUNKS, K), jnp.int32),
          pltpu.VMEM((CHUNKS, K), jnp.int32),
          pltpu.VMEM((CHUNKS, K), jnp.float32),
          pltpu.SemaphoreType.DMA,
      ],
      compiler_params=pltpu.CompilerParams(needs_layout_passes=False),
  )


def _dense_body(x_ref, agg_ref, w1rel_ref, w1root_ref, b1_ref, w2_ref,
                pqt_ref):
  agg = agg_ref[0] + agg_ref[1]
  s = jnp.dot(agg, w1rel_ref[...], preferred_element_type=jnp.float32)
  s += jnp.dot(x_ref[...], w1root_ref[...], preferred_element_type=jnp.float32)
  s += b1_ref[...]
  s = jnp.maximum(s, 0.0)
  pqt_ref[...] = lax.dot_general(
      w2_ref[...], s, (((0,), (1,)), ((), ())),
      preferred_element_type=jnp.float32)


def _final_body(agg2_ref, pqt_ref, b2_ref, out_ref):
  asum = agg2_ref[0, 0]
  for c in range(NC):
    for s2 in range(NS):
      if c or s2:
        asum += agg2_ref[c, s2]
  t = asum + pqt_ref[4:6, :] + b2_ref[...]
  m = jnp.max(t, axis=0, keepdims=True)
  lse = m + jnp.log(jnp.sum(jnp.exp(t - m), axis=0, keepdims=True))
  out_ref[...] = t - lse


def kernel(x, edge_index, edge_attr, W1_rel, b1_rel, W1_root, W2_rel, b2_rel,
           W2_root):
  src = edge_index[0]
  dst = edge_index[1]
  pad = EP - E
  src_r = jnp.pad(src, (0, pad)).reshape(NC, NS, CHUNKS, K)
  dst_r = jnp.pad(dst, (0, pad)).reshape(NC, NS, CHUNKS, K)
  ew_r = jnp.pad(edge_attr, (0, pad)).reshape(NC, NS, CHUNKS, K)
  zrows = jnp.zeros((ROWS_PER_SUBCORE, D), dtype=jnp.float32)
  zcol = jnp.zeros((NP,), dtype=jnp.float32)

  agg1 = _make_seg1()(x, src_r, dst_r, ew_r, zrows)

  w2pack = jnp.zeros((H, 8), dtype=jnp.float32)
  w2pack = w2pack.at[:, :C].set(W2_rel.T)
  w2pack = w2pack.at[:, 4:4 + C].set(W2_root.T)

  pqt = pl.pallas_call(
      _dense_body,
      grid=(GRID,),
      in_specs=[
          pl.BlockSpec((BN, D), lambda i: (i, 0)),
          pl.BlockSpec((NC, BN, D), lambda i: (0, i, 0)),
          pl.BlockSpec((D, H), lambda i: (0, 0)),
          pl.BlockSpec((D, H), lambda i: (0, 0)),
          pl.BlockSpec((1, H), lambda i: (0, 0)),
          pl.BlockSpec((H, 8), lambda i: (0, 0)),
      ],
      out_specs=pl.BlockSpec((8, BN), lambda i: (0, i)),
      out_shape=jax.ShapeDtypeStruct((8, NP), jnp.float32),
  )(x, agg1, W1_rel.T, W1_root.T, b1_rel.reshape(1, H), w2pack)

  agg2 = _make_seg2()(pqt, src_r, dst_r, ew_r, zcol)

  b2col = b2_rel.reshape(C, 1)
  outT = pl.pallas_call(
      _final_body,
      grid=(GRID,),
      in_specs=[
          pl.BlockSpec((NC, NS, C, BN), lambda i: (0, 0, 0, i)),
          pl.BlockSpec((8, BN), lambda i: (0, i)),
          pl.BlockSpec((C, 1), lambda i: (0, 0)),
      ],
      out_specs=pl.BlockSpec((C, BN), lambda i: (0, i)),
      out_shape=jax.ShapeDtypeStruct((C, N), jnp.float32),
  )(agg2, pqt, b2col)
  return outT.T


# speedup vs baseline: 12.4933x; 1.2671x over previous
import functools

import jax
import jax.numpy as jnp
from jax import lax
from jax.experimental import pallas as pl
from jax.experimental.pallas import tpu as pltpu
from jax.experimental.pallas import tpu_sc as plsc

N = 10000
E = 320000
D = 128
H = 512
C = 2

NC = 2
NS = 16
NW = NC * NS
K = 128
CHT = 158
CH0 = 42
CH1 = CHT - CH0
CHC = 79
EP = NS * CHT * K
NP = 10240
ROWS_PER_SUBCORE = NP // NS
BN = 1024
GRID = NP // BN


def _seg1_body(x_hbm, src_hbm, dst_hbm, ew_hbm, zrows_hbm, out_hbm,
               src_v, dst_v, ew_v, rows_v, acc_sh, gsem, isem):
  cid = lax.axis_index("c")
  sid = lax.axis_index("s")
  start = cid * CH0
  cnt = jnp.where(cid == 0, CH0, CH1)

  rsl = pl.ds(sid * ROWS_PER_SUBCORE, ROWS_PER_SUBCORE)
  pltpu.sync_copy(zrows_hbm, acc_sh.at[rsl])
  plsc.subcore_barrier()

  def load_idx(c, b):
    d1 = pltpu.async_copy(src_hbm.at[sid, c], src_v.at[b], isem)
    d2 = pltpu.async_copy(dst_hbm.at[sid, c], dst_v.at[b], isem)
    d3 = pltpu.async_copy(ew_hbm.at[sid, c], ew_v.at[b], isem)
    d1.wait(); d2.wait(); d3.wait()

  def start_gather(b):
    return pltpu.async_copy(x_hbm.at[src_v.at[b]], rows_v.at[b], gsem)

  def process(c, b):
    def group_body(g, carry2):
      ewg = ew_v[b, pl.ds(g * 16, 16)]
      for j in range(16):
        e = g * 16 + j
        w = jnp.full((16,), ewg[j], dtype=jnp.float32)
        for jj in range(D // 16):
          sl = pl.ds(jj * 16, 16)
          rows_v[b, e, sl] = rows_v[b, e, sl] * w
      return carry2

    lax.fori_loop(0, K // 16, group_body, 0)
    pltpu.sync_copy(rows_v.at[b], acc_sh.at[dst_v.at[b]], add=True)

  load_idx(start, 0)
  start_gather(0)

  def pair_body(i, carry):
    for b in range(2):
      j = 2 * i + b
      c = start + j

      @pl.when(j + 1 < cnt)
      def _prefetch():
        load_idx(c + 1, 1 - b)
        start_gather(1 - b)

      pltpu.make_async_copy(x_hbm.at[src_v.at[b]], rows_v.at[b], gsem).wait()
      process(c, b)
    return carry

  lax.fori_loop(0, cnt // 2, pair_body, 0)
  plsc.subcore_barrier()

  pltpu.sync_copy(acc_sh.at[rsl], out_hbm.at[cid, rsl])


def _make_seg1():
  mesh = plsc.VectorSubcoreMesh(core_axis_name="c", subcore_axis_name="s",
                                num_cores=NC, num_subcores=NS)
  return pl.kernel(
      _seg1_body,
      out_type=jax.ShapeDtypeStruct((NC, NP, D), jnp.float32),
      mesh=mesh,
      scratch_types=[
          pltpu.VMEM((2, K), jnp.int32),
          pltpu.VMEM((2, K), jnp.int32),
          pltpu.VMEM((2, K), jnp.float32),
          pltpu.VMEM((2, K, D), jnp.float32),
          pltpu.VMEM_SHARED((NP, D), jnp.float32),
          pltpu.SemaphoreType.DMA,
          pltpu.SemaphoreType.DMA,
      ],
  )


def _seg2_body(pqt_hbm, src_hbm, dst_hbm, ew_hbm, z_hbm, out_hbm,
               p0_v, p1_v, a0_v, a1_v, src_v, dst_v, ew_v, esem):
  cid = lax.axis_index("c")
  sid = lax.axis_index("s")
  d1 = pltpu.async_copy(src_hbm.at[cid, sid], src_v, esem)
  d2 = pltpu.async_copy(dst_hbm.at[cid, sid], dst_v, esem)
  d3 = pltpu.async_copy(ew_hbm.at[cid, sid], ew_v, esem)
  pltpu.sync_copy(pqt_hbm.at[0], p0_v)
  pltpu.sync_copy(pqt_hbm.at[1], p1_v)
  pltpu.sync_copy(z_hbm, a0_v)
  pltpu.sync_copy(z_hbm, a1_v)
  d1.wait(); d2.wait(); d3.wait()

  def chunk_body(c, carry):
    def group_body(g, carry2):
      sl = pl.ds(g * 16, 16)
      sidx = src_v[c, sl]
      didx = dst_v[c, sl]
      eww = ew_v[c, sl]
      for pv, av in ((p0_v, a0_v), (p1_v, a1_v)):
        v = plsc.load_gather(pv, [sidx])
        plsc.addupdate_scatter(av, [didx], v * eww)
      return carry2

    lax.fori_loop(0, K // 16, group_body, 0)
    return carry

  lax.fori_loop(0, CHC, chunk_body, 0)
  pltpu.sync_copy(a0_v, out_hbm.at[cid, sid, 0])
  pltpu.sync_copy(a1_v, out_hbm.at[cid, sid, 1])


def _make_seg2():
  mesh = plsc.VectorSubcoreMesh(core_axis_name="c", subcore_axis_name="s",
                                num_cores=NC, num_subcores=NS)
  return pl.kernel(
      _seg2_body,
      out_type=jax.ShapeDtypeStruct((NC, NS, C, NP), jnp.float32),
      mesh=mesh,
      scratch_types=[
          pltpu.VMEM((NP,), jnp.float32),
          pltpu.VMEM((NP,), jnp.float32),
          pltpu.VMEM((NP,), jnp.float32),
          pltpu.VMEM((NP,), jnp.float32),
          pltpu.VMEM((CHC, K), jnp.int32),
          pltpu.VMEM((CHC, K), jnp.int32),
          pltpu.VMEM((CHC, K), jnp.float32),
          pltpu.SemaphoreType.DMA,
      ],
      compiler_params=pltpu.CompilerParams(needs_layout_passes=False),
  )


def _dense_body(x_ref, agg_ref, w1rel_ref, w1root_ref, b1_ref, w2_ref,
                pqt_ref):
  agg = agg_ref[0] + agg_ref[1]
  s = jnp.dot(agg, w1rel_ref[...], preferred_element_type=jnp.float32)
  s += jnp.dot(x_ref[...], w1root_ref[...], preferred_element_type=jnp.float32)
  s += b1_ref[...]
  s = jnp.maximum(s, 0.0)
  pqt_ref[...] = lax.dot_general(
      w2_ref[...], s, (((0,), (1,)), ((), ())),
      preferred_element_type=jnp.float32)


def _final_body(agg2_ref, pqt_ref, b2_ref, out_ref):
  asum = agg2_ref[0, 0]
  for c in range(NC):
    for s2 in range(NS):
      if c or s2:
        asum += agg2_ref[c, s2]
  t = asum + pqt_ref[4:6, :] + b2_ref[...]
  m = jnp.max(t, axis=0, keepdims=True)
  lse = m + jnp.log(jnp.sum(jnp.exp(t - m), axis=0, keepdims=True))
  out_ref[...] = t - lse


def kernel(x, edge_index, edge_attr, W1_rel, b1_rel, W1_root, W2_rel, b2_rel,
           W2_root):
  src = edge_index[0]
  dst = edge_index[1]
  pad = EP - E
  srcp = jnp.pad(src, (0, pad))
  dstp = jnp.pad(dst, (0, pad))
  ewp = jnp.pad(edge_attr, (0, pad))
  src_a = srcp.reshape(NS, CHT, K)
  dst_a = dstp.reshape(NS, CHT, K)
  ew_a = ewp.reshape(NS, CHT, K)
  src_c = srcp.reshape(NC, NS, CHC, K)
  dst_c = dstp.reshape(NC, NS, CHC, K)
  ew_c = ewp.reshape(NC, NS, CHC, K)
  zrows = jnp.zeros((ROWS_PER_SUBCORE, D), dtype=jnp.float32)
  zcol = jnp.zeros((NP,), dtype=jnp.float32)

  agg1 = _make_seg1()(x, src_a, dst_a, ew_a, zrows)

  w2pack = jnp.zeros((H, 8), dtype=jnp.float32)
  w2pack = w2pack.at[:, :C].set(W2_rel.T)
  w2pack = w2pack.at[:, 4:4 + C].set(W2_root.T)

  pqt = pl.pallas_call(
      _dense_body,
      grid=(GRID,),
      in_specs=[
          pl.BlockSpec((BN, D), lambda i: (i, 0)),
          pl.BlockSpec((NC, BN, D), lambda i: (0, i, 0)),
          pl.BlockSpec((D, H), lambda i: (0, 0)),
          pl.BlockSpec((D, H), lambda i: (0, 0)),
          pl.BlockSpec((1, H), lambda i: (0, 0)),
          pl.BlockSpec((H, 8), lambda i: (0, 0)),
      ],
      out_specs=pl.BlockSpec((8, BN), lambda i: (0, i)),
      out_shape=jax.ShapeDtypeStruct((8, NP), jnp.float32),
  )(x, agg1, W1_rel.T, W1_root.T, b1_rel.reshape(1, H), w2pack)

  agg2 = _make_seg2()(pqt, src_c, dst_c, ew_c, zcol)

  b2col = b2_rel.reshape(C, 1)
  outT = pl.pallas_call(
      _final_body,
      grid=(GRID,),
      in_specs=[
          pl.BlockSpec((NC, NS, C, BN), lambda i: (0, 0, 0, i)),
          pl.BlockSpec((8, BN), lambda i: (0, i)),
          pl.BlockSpec((C, 1), lambda i: (0, 0)),
      ],
      out_specs=pl.BlockSpec((C, BN), lambda i: (0, i)),
      out_shape=jax.ShapeDtypeStruct((C, N), jnp.float32),
  )(agg2, pqt, b2col)
  return outT.T
